# Initial kernel scaffold; baseline (speedup 1.0000x reference)
import jax
import jax.numpy as jnp
from jax.experimental import pallas as pl


def kernel(idx, token_table, pos_table):
    raise NotImplementedError("write your pallas kernel here")


# speedup vs baseline: 5.2090x; 5.2090x over previous
import functools

import jax
import jax.numpy as jnp
from jax import lax
from jax.experimental import pallas as pl
from jax.experimental.pallas import tpu as pltpu
from jax.experimental.pallas import tpu_sc as plsc

_D = 32
_C = 1024


def kernel(idx, token_table, pos_table):
    B, T = idx.shape
    N = B * T
    NC, NS = 2, 16
    NW = NC * NS
    per_w = N // NW
    n_chunks = per_w // _C

    mesh = plsc.VectorSubcoreMesh(core_axis_name="c", subcore_axis_name="s")

    @functools.partial(
        pl.kernel,
        mesh=mesh,
        out_type=jax.ShapeDtypeStruct((N, _D), jnp.float32),
        scratch_types=[
            pltpu.VMEM((_C,), jnp.int32),
            pltpu.VMEM((_C, _D), jnp.float32),
            pltpu.VMEM((T, _D), jnp.float32),
            pltpu.SemaphoreType.DMA,
        ],
        compiler_params=pltpu.CompilerParams(use_tc_tiling_on_sc=False),
    )
    def k(idx_hbm, table_hbm, out_hbm, idx_v, rows_v, pos_v, sem):
        wid = lax.axis_index("s") * NC + lax.axis_index("c")
        base = wid * per_w
        pltpu.sync_copy(table_hbm.at[pl.ds(0, T)], pos_v)

        def chunk_body(g, carry):
            off = base + g * _C
            pltpu.sync_copy(idx_hbm.at[pl.ds(off, _C)], idx_v)
            pltpu.async_copy(table_hbm.at[idx_v], rows_v, sem).wait()

            def j_body(j, c2):
                p0 = pos_v[j, pl.ds(0, 16)]
                p1 = pos_v[j, pl.ds(16, 16)]
                for rep in range(_C // T):
                    r = rep * T + j
                    rows_v[r, pl.ds(0, 16)] = rows_v[r, pl.ds(0, 16)] + p0
                    rows_v[r, pl.ds(16, 16)] = rows_v[r, pl.ds(16, 16)] + p1
                return c2

            lax.fori_loop(0, T, j_body, 0)
            pltpu.sync_copy(rows_v, out_hbm.at[pl.ds(off, _C)])
            return carry

        lax.fori_loop(0, n_chunks, chunk_body, 0)

    out = k(idx.reshape(-1), token_table)
    return out.reshape(B, T, _D)
